# baseline (speedup 1.0000x reference)
import functools

import jax
import jax.numpy as jnp
from jax import lax
from jax.experimental import pallas as pl
from jax.experimental.pallas import tpu as pltpu
from jax.experimental.pallas import tpu_sc as plsc

NUM_FEATURES = 64
NUM_SAMPLES = 100000
NUM_CLASSES = 751
TEMP = 0.05
B = 1024

NC = 2
NS = 16
NW = NC * NS

CHUNK = 128
FULL_CHUNKS = NUM_SAMPLES // CHUNK
TAIL = NUM_SAMPLES - FULL_CHUNKS * CHUNK
KMAIN = FULL_CHUNKS // NW
EXTRA = FULL_CHUNKS - KMAIN * NW
ROWS_PAD = 768
GW = ROWS_PAD * NUM_FEATURES
CPAD = 768
CROW = 756
RSTRIPE = ROWS_PAD // NS
IDX_PER_W = B // NW


def _iota16():
    return lax.iota(jnp.int32, 16)


def _sc_body(feat_hbm, lab_hbm, idx_hbm, g_out, t_out,
             featv0, featv1, ints, g_bank, stage_g,
             semf0, semf1, seml0, seml1, semt0, semt1, semx):
    cid = lax.axis_index("c")
    sid = lax.axis_index("s")
    w = sid * NC + cid
    it = _iota16()
    featb = (featv0, featv1)
    labo = (0, CHUNK)
    semfb = (semf0, semf1)
    semlb = (seml0, seml1)

    def chunk_row(k):
        return pl.multiple_of((w + NW * k) * CHUNK, CHUNK)

    def ldst(b, n=CHUNK):
        return ints.at[pl.ds(labo[b], n)]

    for b in range(2):
        r = chunk_row(b)
        pltpu.async_copy(feat_hbm.at[pl.ds(r, CHUNK)], featb[b], semfb[b])
        pltpu.async_copy(lab_hbm.at[pl.ds(r, CHUNK)], ldst(b), semlb[b])

    zf = jnp.zeros((16,), jnp.float32)
    cols = [j * 16 + it for j in range(NUM_FEATURES // 16)]

    @plsc.parallel_loop(0, ROWS_PAD, unroll=2)
    def _(k):
        rk = jnp.full((16,), k, jnp.int32)
        for q in range(NUM_FEATURES // 16):
            plsc.store_scatter(g_bank, [rk, cols[q]], zf)

    one16 = jnp.ones((16,), jnp.float32)
    lane0 = it == 0

    def sample(fv, lo, i):
        rowi = jnp.full((16,), i, jnp.int32)
        lblv = plsc.load_gather(ints, [lo + rowi])
        for j in range(NUM_FEATURES // 16):
            row = plsc.load_gather(fv, [rowi, cols[j]])
            plsc.addupdate_scatter(g_bank, [lblv, cols[j]], row)
        plsc.addupdate_scatter(
            g_bank, [CROW + lax.shift_right_logical(lblv, 6), lblv & 63],
            one16, mask=lane0)

    def accumulate(b, nsamples):
        fv, lo = featb[b], labo[b]

        @plsc.parallel_loop(0, nsamples // 4, unroll=2)
        def _(q):
            for u in range(4):
                sample(fv, lo, q * 4 + u)

    def cbody(k2, carry):
        for b in range(2):
            k = k2 * 2 + b
            r = chunk_row(k)
            pltpu.make_async_copy(feat_hbm.at[pl.ds(r, CHUNK)], featb[b], semfb[b]).wait()
            pltpu.make_async_copy(lab_hbm.at[pl.ds(r, CHUNK)], ldst(b), semlb[b]).wait()
            accumulate(b, CHUNK)

            @pl.when(k + 2 < KMAIN)
            def _():
                rn = chunk_row(k + 2)
                pltpu.async_copy(feat_hbm.at[pl.ds(rn, CHUNK)], featb[b], semfb[b])
                pltpu.async_copy(lab_hbm.at[pl.ds(rn, CHUNK)], ldst(b), semlb[b])
        return carry
    lax.fori_loop(0, KMAIN // 2, cbody, 0)

    @pl.when(w < EXTRA)
    def _():
        r = pl.multiple_of((KMAIN * NW + w) * CHUNK, CHUNK)
        pltpu.async_copy(feat_hbm.at[pl.ds(r, CHUNK)], featv0, semf0)
        pltpu.async_copy(lab_hbm.at[pl.ds(r, CHUNK)], ldst(0), seml0)
        pltpu.make_async_copy(feat_hbm.at[pl.ds(r, CHUNK)], featv0, semf0).wait()
        pltpu.make_async_copy(lab_hbm.at[pl.ds(r, CHUNK)], ldst(0), seml0).wait()
        accumulate(0, CHUNK)

    @pl.when(w == EXTRA)
    def _():
        s = feat_hbm.at[pl.ds(FULL_CHUNKS * CHUNK, TAIL)]
        sl = lab_hbm.at[pl.ds(FULL_CHUNKS * CHUNK, TAIL)]
        fd = featv0.at[pl.ds(0, TAIL)]
        ld = ldst(0, TAIL)
        pltpu.async_copy(s, fd, semf0)
        pltpu.async_copy(sl, ld, seml0)
        pltpu.make_async_copy(s, fd, semf0).wait()
        pltpu.make_async_copy(sl, ld, seml0).wait()
        accumulate(0, TAIL)

    ibase = pl.multiple_of(w * IDX_PER_W, IDX_PER_W)
    pltpu.sync_copy(idx_hbm.at[pl.ds(ibase, IDX_PER_W)], ints.at[pl.ds(256, IDX_PER_W)])
    for j in range(IDX_PER_W // 16):
        v = ints[pl.ds(256 + j * 16, 16)] - 1
        v = jnp.where(v >= 0, v, 0)
        v = jnp.where(v == 5554, NUM_CLASSES - 1, v)
        ints[pl.ds(288 + j * 16, 16)] = v
    pltpu.async_copy(lab_hbm.at[ints.at[pl.ds(288, IDX_PER_W)]],
                     ints.at[pl.ds(320, IDX_PER_W)], semx).wait()
    pltpu.sync_copy(ints.at[pl.ds(320, IDX_PER_W)], t_out.at[pl.ds(ibase, IDX_PER_W)])

    pltpu.sync_copy(g_bank, stage_g.at[pl.ds(pl.multiple_of(sid * ROWS_PAD, 128), ROWS_PAD)])
    plsc.subcore_barrier()

    semtb = (semt0, semt1)
    gr = sid * RSTRIPE

    def gsl(b):
        return stage_g.at[pl.ds(pl.multiple_of(b * ROWS_PAD + gr, 8), RSTRIPE)]

    def fsl(b):
        return featb[b].at[pl.ds(0, RSTRIPE)]
    accv = g_bank.at[pl.ds(0, RSTRIPE)]
    pltpu.async_copy(gsl(0), accv, semx)
    pltpu.async_copy(gsl(1), fsl(0), semt0)
    pltpu.make_async_copy(gsl(0), accv, semx).wait()
    for b in range(1, NS):
        p = (b - 1) % 2
        pltpu.make_async_copy(gsl(b), fsl(p), semtb[p]).wait()
        if b + 1 < NS:
            pltpu.async_copy(gsl(b + 1), fsl(b % 2), semtb[b % 2])
        tv = featb[p]

        @plsc.parallel_loop(0, RSTRIPE, unroll=2)
        def _(m):
            rm = jnp.full((16,), m, jnp.int32)
            for q in range(NUM_FEATURES // 16):
                acc = (plsc.load_gather(g_bank, [rm, cols[q]])
                       + plsc.load_gather(tv, [rm, cols[q]]))
                plsc.store_scatter(g_bank, [rm, cols[q]], acc)

    pltpu.sync_copy(accv, g_out.at[pl.ds(pl.multiple_of(cid * ROWS_PAD + gr, 8), RSTRIPE)])


@functools.cache
def _make_sc_seg():
    return pl.kernel(
        _sc_body,
        out_type=(
            jax.ShapeDtypeStruct((NC * ROWS_PAD, NUM_FEATURES), jnp.float32),
            jax.ShapeDtypeStruct((B,), jnp.int32),
        ),
        compiler_params=pltpu.CompilerParams(needs_layout_passes=False,
                                             use_tc_tiling_on_sc=False),
        mesh=plsc.VectorSubcoreMesh(core_axis_name="c", subcore_axis_name="s",
                                    num_cores=NC, num_subcores=NS),
        scratch_types=[
            pltpu.VMEM((CHUNK, NUM_FEATURES), jnp.float32),
            pltpu.VMEM((CHUNK, NUM_FEATURES), jnp.float32),
            pltpu.VMEM((2 * CHUNK + 3 * IDX_PER_W,), jnp.int32),
            pltpu.VMEM((ROWS_PAD, NUM_FEATURES), jnp.float32),
            pltpu.VMEM_SHARED((NS * ROWS_PAD, NUM_FEATURES), jnp.float32),
            pltpu.SemaphoreType.DMA,
            pltpu.SemaphoreType.DMA,
            pltpu.SemaphoreType.DMA,
            pltpu.SemaphoreType.DMA,
            pltpu.SemaphoreType.DMA,
            pltpu.SemaphoreType.DMA,
            pltpu.SemaphoreType.DMA,
        ],
    )


def _tc_body(x_ref, idx_ref, tgt_ref, g_ref, c_ref, o_ref):
    x = x_ref[...]
    nrm = jnp.sqrt(jnp.sum(x * x, axis=1, keepdims=True))
    x = x / jnp.clip(nrm, 1e-12, None)
    g = (g_ref[0] + g_ref[1])[:NUM_CLASSES]
    cnt_row = (c_ref[0] + c_ref[1])[:NUM_CLASSES].reshape(1, NUM_CLASSES)
    sim = lax.dot_general(x, g, (((1,), (1,)), ((), ())),
                          preferred_element_type=jnp.float32)
    pos = cnt_row > 0
    scale = jnp.where(pos, 1.0 / (TEMP * cnt_row), 0.0)
    exps = jnp.exp(sim * scale) * pos.astype(jnp.float32)
    sums = jnp.sum(exps, axis=1, keepdims=True) + 1e-6
    logp = jnp.log(exps / sums + 1e-6)
    tgt = tgt_ref[...]
    cols = lax.broadcasted_iota(jnp.int32, (B, NUM_CLASSES), 1)
    picked = jnp.sum(jnp.where(cols == tgt, logp, 0.0), axis=1, keepdims=True)
    keep = (idx_ref[...] - 1) >= 0
    valid = jnp.logical_and(tgt != NUM_CLASSES - 1, keep).astype(jnp.float32)
    num = jnp.sum(picked * valid)
    den = jnp.maximum(jnp.sum(valid), 1.0)
    o_ref[...] = jnp.broadcast_to(-num / den, (1, 1))


def kernel(inputs, indexes, features, labels):
    idx32 = indexes.astype(jnp.int32)
    g_parts, targets = _make_sc_seg()(
        features, labels.astype(jnp.int32), idx32)
    g_parts = g_parts.reshape(NC, ROWS_PAD, NUM_FEATURES)
    c_parts = g_parts[:, CROW:, :].reshape(NC, CPAD)
    loss = pl.pallas_call(
        _tc_body,
        out_shape=jax.ShapeDtypeStruct((1, 1), jnp.float32),
    )(inputs, idx32.reshape(B, 1), targets.reshape(B, 1), g_parts, c_parts)
    return loss[0, 0]
